# Initial kernel scaffold; baseline (speedup 1.0000x reference)
import jax
import jax.numpy as jnp
from jax.experimental import pallas as pl


def kernel(edge_rep, cycle_rep, e5_idx, e6_idx, W_lift, W_lvl1, W_lvl2, eps1_1, eps1_2, eps2):
    raise NotImplementedError("write your pallas kernel here")


# speedup vs baseline: 1.4932x; 1.4932x over previous
import functools

import jax
import jax.numpy as jnp
from jax import lax
from jax.experimental import pallas as pl
from jax.experimental.pallas import tpu as pltpu
from jax.experimental.pallas import tpu_sc as plsc

E = 160000
C5 = 12000
C6 = 12000
H = 128
N5 = 5 * C5
N6 = 6 * C6
N = N5 + N6

NC, NS = 2, 16
NW = NC * NS

_mesh = plsc.VectorSubcoreMesh(core_axis_name="c", subcore_axis_name="s",
                               num_cores=NC, num_subcores=NS)
_sc_params = pltpu.CompilerParams(needs_layout_passes=False)

CHB = 2000
NBKTB = E // CHB
J = 2 * N
JP = 266112
PER_W = 8316
PER_W = 8256
JP = NW * PER_W
CAP_W = PER_W + 81 * 15
CAP_W = ((CAP_W + 15) // 16) * 16
BINSZ = NW * CAP_W
BIN_TAIL = 192
B2B = 160
B2V = B2B // 16


def _wid():
    return lax.axis_index("s") * NC + lax.axis_index("c")


def _cycle_phase(src_hbm, idx_hbm, out_hbm, vg, vout, vidx, sem, *,
                 L, ncyc, n_units, atom_off, width, do_sums, wid):
    AU = L * ncyc
    u_lo = wid * n_units // NW
    u_hi = (wid + 1) * n_units // NW

    def unit_body(u, carry):
        a0 = pl.multiple_of(atom_off + u * AU, 8)
        pltpu.sync_copy(idx_hbm.at[pl.ds(a0, AU)], vidx)
        pltpu.async_copy(src_hbm.at[vidx], vg, sem).wait()

        def cyc_body(jc, c2):
            r0 = jc * L
            for k in range(width // 16):
                sl = pl.ds(k * 16, 16)
                g = [vg[r0 + i, sl] for i in range(L)]
                if do_sums:
                    s = g[0] + g[1]
                    for i in range(2, L):
                        s = s + g[i]
                    s = s * 2.0
                    so = pl.ds(width + k * 16, 16)
                    for i in range(L):
                        vout[r0 + i, so] = s
                for i in range(L):
                    x = g[i] + g[(i - 1) % L]
                    if do_sums:
                        vout[r0 + i, sl] = x
                    else:
                        vg[r0 + i, sl] = x
            return c2

        lax.fori_loop(0, ncyc, cyc_body, 0)
        out_src = vout if do_sums else vg
        pltpu.sync_copy(out_src, out_hbm.at[pl.ds(a0, AU)])
        return carry

    lax.fori_loop(u_lo, u_hi, unit_body, 0)


@functools.partial(
    pl.kernel, mesh=_mesh,
    out_type=jax.ShapeDtypeStruct((N, 2 * H), jnp.float32),
    scratch_types=[
        pltpu.VMEM((240, H), jnp.float32),
        pltpu.VMEM((240, 2 * H), jnp.float32),
        pltpu.VMEM((240,), jnp.int32),
        pltpu.SemaphoreType.DMA,
    ],
    compiler_params=_sc_params,
)
def _kernel_a(edge_hbm, cur_hbm, lift_hbm, vg, vout, vidx, sem):
    wid = _wid()
    _cycle_phase(edge_hbm, cur_hbm, lift_hbm, vg, vout, vidx, sem,
                 L=5, ncyc=48, n_units=C5 // 48, atom_off=0,
                 width=H, do_sums=True, wid=wid)
    _cycle_phase(edge_hbm, cur_hbm, lift_hbm, vg, vout, vidx, sem,
                 L=6, ncyc=40, n_units=C6 // 40, atom_off=N5,
                 width=H, do_sums=True, wid=wid)


@functools.partial(
    pl.kernel, mesh=_mesh,
    out_type=(jax.ShapeDtypeStruct((N, H), jnp.float32),
              jax.ShapeDtypeStruct((N, H), jnp.float32)),
    scratch_types=[
        pltpu.VMEM((240, H), jnp.float32),
        pltpu.VMEM((240, H), jnp.float32),
        pltpu.VMEM((240,), jnp.int32),
        pltpu.SemaphoreType.DMA,
        pltpu.SemaphoreType.DMA,
    ],
    compiler_params=_sc_params,
)
def _kernel_c(ilo_hbm, ihi_hbm, cur_hbm, lmlo_hbm, lmhi_hbm,
              vglo, vghi, vidx, semlo, semhi):
    wid = _wid()
    for args in ((ilo_hbm, lmlo_hbm, vglo, semlo),
                 (ihi_hbm, lmhi_hbm, vghi, semhi)):
        src, out, vg, sem = args
        _cycle_phase(src, cur_hbm, out, vg, None, vidx, sem,
                     L=5, ncyc=48, n_units=C5 // 48, atom_off=0,
                     width=H, do_sums=False, wid=wid)
        _cycle_phase(src, cur_hbm, out, vg, None, vidx, sem,
                     L=6, ncyc=40, n_units=C6 // 40, atom_off=N5,
                     width=H, do_sums=False, wid=wid)


@functools.partial(
    pl.kernel, mesh=_mesh,
    out_type=(
        jax.ShapeDtypeStruct((BINSZ + BIN_TAIL,), jnp.int32),
        jax.ShapeDtypeStruct((BINSZ + BIN_TAIL,), jnp.int32),
        jax.ShapeDtypeStruct((NW, 96), jnp.int32),
        jax.ShapeDtypeStruct((NW, 96), jnp.int32),
    ),
    scratch_types=[
        pltpu.VMEM((PER_W,), jnp.int32),
        pltpu.VMEM((CAP_W,), jnp.int32),
        pltpu.VMEM((CAP_W,), jnp.int32),
        pltpu.VMEM((96,), jnp.int32),
        pltpu.VMEM((96,), jnp.int32),
        pltpu.VMEM((96,), jnp.int32),
        pltpu.VMEM((96,), jnp.int32),
    ],
    compiler_params=_sc_params,
)
def _kernel_b1(dbl_hbm, dep_hbm, bsrc_hbm, btgt_hbm, tstart_hbm, tnv_hbm,
               vtg, bsrc, btgt, hist, off, gstart, nv16):
    del dep_hbm
    wid = _wid()
    j0 = pl.multiple_of(wid * PER_W, 8)
    pltpu.sync_copy(dbl_hbm.at[pl.ds(j0, PER_W)], vtg)
    iota = lax.iota(jnp.int32, 16)
    zero16 = jnp.zeros((16,), jnp.int32)
    for k in range(6):
        hist[pl.ds(k * 16, 16)] = zero16

    eful = jnp.full((16,), E, jnp.int32)

    def fill_body(v, carry):
        bsrc[pl.ds(v * 16, 16)] = iota + v * 16
        btgt[pl.ds(v * 16, 16)] = eful
        return carry

    lax.fori_loop(0, CAP_W // 16, fill_body, 0)

    def hist_body(v, carry):
        t = vtg[pl.ds(v * 16, 16)]
        b = t // CHB
        cnt, last = plsc.scan_count(b)
        plsc.addupdate_scatter(hist, [b], cnt, mask=last)
        return carry

    lax.fori_loop(0, PER_W // 16, hist_body, 0)

    carry = jnp.zeros((), jnp.int32)
    for k in range(6):
        h = hist[pl.ds(k * 16, 16)]
        hp = ((h + 15) // 16) * 16
        inc = plsc.cumsum(hp)
        exc = inc - hp + carry
        off[pl.ds(k * 16, 16)] = exc
        gstart[pl.ds(k * 16, 16)] = exc + wid * CAP_W
        nv16[pl.ds(k * 16, 16)] = (h + 15) // 16
        carry = inc[15] + carry

    pltpu.sync_copy(gstart, tstart_hbm.at[wid])
    pltpu.sync_copy(nv16, tnv_hbm.at[wid])

    def bin_body(v, carry2):
        t = vtg[pl.ds(v * 16, 16)]
        b = t // CHB
        cnt, last = plsc.scan_count(b)
        o = plsc.load_gather(off, [b])
        pos = o + cnt - 1
        jg = j0 + v * 16 + iota
        src = jnp.where(jg >= N, jg - N, jg)
        plsc.store_scatter(bsrc, [pos], src)
        plsc.store_scatter(btgt, [pos], t)
        plsc.store_scatter(off, [b], pos + 1, mask=last)
        return carry2

    lax.fori_loop(0, PER_W // 16, bin_body, 0)

    for k in range(6):
        bid = iota + k * 16
        ends = off[pl.ds(k * 16, 16)]
        h = hist[pl.ds(k * 16, 16)]
        npad = (16 - (h & 15)) & 15
        dsrc = (bid * 7 + wid * 53) % N
        dtgt = jnp.full((16,), E, jnp.int32)
        for p in range(15):
            m = npad > p
            plsc.store_scatter(bsrc, [ends + p], dsrc, mask=m)
            plsc.store_scatter(btgt, [ends + p], dtgt, mask=m)

    wcap = pl.multiple_of(wid * CAP_W, 8)
    pltpu.sync_copy(bsrc, bsrc_hbm.at[pl.ds(wcap, CAP_W)])
    pltpu.sync_copy(btgt, btgt_hbm.at[pl.ds(wcap, CAP_W)])

    @pl.when(wid == NW - 1)
    def _():
        for v in range(BIN_TAIL // 16):
            bsrc[pl.ds(v * 16, 16)] = iota + v * 16
            btgt[pl.ds(v * 16, 16)] = jnp.full((16,), E, jnp.int32)
        pltpu.sync_copy(bsrc.at[pl.ds(0, BIN_TAIL)],
                        bsrc_hbm.at[pl.ds(BINSZ, BIN_TAIL)])
        pltpu.sync_copy(btgt.at[pl.ds(0, BIN_TAIL)],
                        btgt_hbm.at[pl.ds(BINSZ, BIN_TAIL)])


def _make_b2(ntab, chp, acc_rows, trows, bpc):
    nchunk = E // chp
    width = H

    @functools.partial(
        pl.kernel, mesh=_mesh,
        out_type=tuple(jax.ShapeDtypeStruct((E, width), jnp.float32)
                       for _ in range(ntab)),
        scratch_types=(
            [pltpu.VMEM((B2B,), jnp.int32)] * 3
            + [pltpu.VMEM((B2B, width), jnp.float32)] * ntab
            + [pltpu.VMEM((128, width), jnp.float32)]
            + [pltpu.VMEM((NW, 96), jnp.int32)] * 2
            + [pltpu.VMEM_SHARED((acc_rows, width), jnp.float32)] * ntab
            + [pltpu.SemaphoreType.DMA] * (1 + ntab)
        ),
        compiler_params=_sc_params,
    )
    def b2(*args):
        (bsrc_hbm, btgt_hbm, tstart_hbm, tnv_hbm, dep_hbm), rest = (
            args[:5], args[5:])
        del dep_hbm
        srcs, rest = rest[:ntab], rest[ntab:]
        outs, rest = rest[:ntab], rest[ntab:]
        (msrc, mtgt, loc), rest = rest[:3], rest[3:]
        rows, rest = rest[:ntab], rest[ntab:]
        (zbuf, vts, vtn), rest = rest[:3], rest[3:]
        accs, rest = rest[:ntab], rest[ntab:]
        semg, sems = rest[0], rest[1:]
        sc = lax.axis_index("c")
        t = lax.axis_index("s")
        iota = lax.iota(jnp.int32, 16)
        zf = jnp.zeros((16,), jnp.float32)
        for r in range(128):
            for kk in range(width // 16):
                zbuf[r, pl.ds(kk * 16, 16)] = zf
        pltpu.sync_copy(tstart_hbm, vts)
        pltpu.sync_copy(tnv_hbm, vtn)

        r0 = pl.multiple_of(t * trows, 8)

        def zero_acc():
            for a in accs:
                for off in range(0, trows, 128):
                    pltpu.sync_copy(zbuf, a.at[pl.ds(r0 + off, 128)])

        zero_acc()
        plsc.subcore_barrier()

        def chunk_body(it, carry):
            c = it * NC + sc
            base = pl.multiple_of(c * chp, 8)
            b0 = c * bpc
            for wsub in range(2):
                w = t * 2 + wsub
                sv = vts[w, pl.ds(b0, 16)]
                nvv = vtn[w, pl.ds(b0, 16)]
                start = sv[0]
                if bpc == 2:
                    nv = (sv[1] - sv[0]) // 16 + nvv[1]
                else:
                    nv = nvv[0]
                nb = (nv + (B2V - 1)) // B2V

                def batch_body(kb, bc):
                    pos = pl.multiple_of(start + kb * B2B, 8)
                    rem = nv * 16 - kb * B2B
                    pltpu.sync_copy(bsrc_hbm.at[pl.ds(pos, B2B)], msrc)
                    pltpu.sync_copy(btgt_hbm.at[pl.ds(pos, B2B)], mtgt)
                    for v in range(B2V):
                        tg = mtgt[pl.ds(v * 16, 16)]
                        lo = tg - base
                        ok = (lo >= 0) & (lo < chp) & ((iota + v * 16) < rem)
                        loc[pl.ds(v * 16, 16)] = jnp.where(ok, lo, chp)
                    for n in range(ntab):
                        pltpu.async_copy(srcs[n].at[msrc], rows[n], semg).wait()
                        pltpu.async_copy(rows[n], accs[n].at[loc],
                                         sems[n], add=True).wait()
                    return bc

                lax.fori_loop(0, nb, batch_body, 0)
            plsc.subcore_barrier()

            for n in range(ntab):
                @pl.when(t < NS - 1)
                def _(n=n):
                    pltpu.sync_copy(accs[n].at[pl.ds(r0, trows)],
                                    outs[n].at[pl.ds(base + r0, trows)])

                @pl.when(t == NS - 1)
                def _(n=n):
                    last = chp - (NS - 1) * trows
                    pltpu.sync_copy(accs[n].at[pl.ds(r0, last)],
                                    outs[n].at[pl.ds(base + r0, last)])

            zero_acc()
            plsc.subcore_barrier()
            return carry

        lax.fori_loop(0, nchunk // NC, chunk_body, 0)

    return b2


_B2Y = _make_b2(1, 4000, 4096, 256, 2)
_B2I = _make_b2(2, 2000, 2048, 128, 1)


def _m1_body(l_ref, clo_ref, chi_ref, w_ref, o_ref):
    w = w_ref[...]
    acc = jnp.dot(l_ref[...], w[0:2 * H, :], preferred_element_type=jnp.float32)
    acc += jnp.dot(clo_ref[...], w[2 * H:3 * H, :],
                   preferred_element_type=jnp.float32)
    acc += jnp.dot(chi_ref[...], w[3 * H:, :],
                   preferred_element_type=jnp.float32)
    o_ref[...] = jnp.maximum(acc, 0.0)


def _m2_body(a_ref, b_ref, w_ref, e1_ref, e2_ref, o_ref):
    x = a_ref[...] * (1.0 + e1_ref[0, 0]) + b_ref[...] * (1.0 + e2_ref[0, 0])
    acc = jnp.dot(x, w_ref[...], preferred_element_type=jnp.float32)
    o_ref[...] = jnp.maximum(acc, 0.0)


def _m3_body(lmlo_ref, lmhi_ref, lf_ref, w_ref, e_ref, o_ref):
    w = w_ref[...]
    lf = lf_ref[...]
    g = 1.0 + e_ref[0, 0]
    xlo = lmlo_ref[...] * g + lf[:, 0:H]
    xhi = lmhi_ref[...] * g + lf[:, H:]
    acc = jnp.dot(xlo, w[0:H, :], preferred_element_type=jnp.float32)
    acc += jnp.dot(xhi, w[H:, :], preferred_element_type=jnp.float32)
    o_ref[...] = jnp.maximum(acc, 0.0)


def _row_spec(rows, cols):
    return pl.BlockSpec((rows, cols), lambda i: (i, 0))


def _full_spec(r, c):
    return pl.BlockSpec((r, c), lambda i: (0, 0))


_M1 = pl.pallas_call(
    _m1_body,
    grid=(N // 528,),
    in_specs=[_row_spec(528, 2 * H), _row_spec(528, H), _row_spec(528, H),
              _full_spec(4 * H, H)],
    out_specs=_row_spec(528, H),
    out_shape=jax.ShapeDtypeStruct((N, H), jnp.float32),
)

_M2 = pl.pallas_call(
    _m2_body,
    grid=(E // 640,),
    in_specs=[_row_spec(640, H), _row_spec(640, H), _full_spec(H, H),
              _full_spec(1, 1), _full_spec(1, 1)],
    out_specs=_row_spec(640, H),
    out_shape=jax.ShapeDtypeStruct((E, H), jnp.float32),
)

_M3 = pl.pallas_call(
    _m3_body,
    grid=(N // 528,),
    in_specs=[_row_spec(528, H), _row_spec(528, H), _row_spec(528, 2 * H),
              _full_spec(2 * H, 2 * H), _full_spec(1, 1)],
    out_specs=_row_spec(528, 2 * H),
    out_shape=jax.ShapeDtypeStruct((N, 2 * H), jnp.float32),
)


def kernel(edge_rep, cycle_rep, e5_idx, e6_idx, W_lift, W_lvl1, W_lvl2,
           eps1_1, eps1_2, eps2):
    e5 = e5_idx.astype(jnp.int32)
    e6 = e6_idx.astype(jnp.int32)
    cur = jnp.concatenate([e5.reshape(-1), e6.reshape(-1)])
    prev = jnp.concatenate([jnp.roll(e5, 1, axis=1).reshape(-1),
                            jnp.roll(e6, 1, axis=1).reshape(-1)])
    dbl = jnp.concatenate([cur, prev,
                           jnp.full((JP - J,), E, jnp.int32)])

    cr_lo = cycle_rep[:, :H]
    cr_hi = cycle_rep[:, H:]
    lift = _kernel_a(edge_rep, cur)
    y = _M1(lift, cr_lo, cr_hi, W_lvl1)
    bsrc, btgt, tstart, tnv = _kernel_b1(dbl, lift[0:8, 0:8])
    (lvl,) = _B2Y(bsrc, btgt, tstart, tnv, y[0:8, 0:8], y)
    ilo, ihi = _B2I(bsrc, btgt, tstart, tnv, lvl[0:8, 0:8], cr_lo, cr_hi)
    lm_lo, lm_hi = _kernel_c(ilo, ihi, cur)
    e11 = eps1_1.reshape(1, 1).astype(jnp.float32)
    e12 = eps1_2.reshape(1, 1).astype(jnp.float32)
    e2 = eps2.reshape(1, 1).astype(jnp.float32)
    edge_out = _M2(edge_rep, lvl, W_lvl2, e11, e12)
    cycle_out = _M3(lm_lo, lm_hi, lift, W_lift, e2)
    return (edge_out, cycle_out)


# speedup vs baseline: 1.6820x; 1.1265x over previous
import functools

import jax
import jax.numpy as jnp
from jax import lax
from jax.experimental import pallas as pl
from jax.experimental.pallas import tpu as pltpu
from jax.experimental.pallas import tpu_sc as plsc

E = 160000
C5 = 12000
C6 = 12000
H = 128
N5 = 5 * C5
N6 = 6 * C6
N = N5 + N6

NC, NS = 2, 16
NW = NC * NS

_mesh = plsc.VectorSubcoreMesh(core_axis_name="c", subcore_axis_name="s",
                               num_cores=NC, num_subcores=NS)
_sc_params = pltpu.CompilerParams(needs_layout_passes=False)

CHB = 2000
NBKTB = E // CHB
J = 2 * N
JP = 266112
PER_W = 8316
PER_W = 8256
JP = NW * PER_W
CAP_W = PER_W + 81 * 15
CAP_W = ((CAP_W + 15) // 16) * 16
BINSZ = NW * CAP_W
BIN_TAIL = 192
B2B = 160
B2V = B2B // 16


def _wid():
    return lax.axis_index("s") * NC + lax.axis_index("c")


def _cycle_phase(src_hbm, idx_hbm, out_hbm, vg, vout, vidx, sem, *,
                 L, ncyc, n_units, atom_off, width, do_sums, wid):
    AU = L * ncyc
    u_lo = wid * n_units // NW
    u_hi = (wid + 1) * n_units // NW

    def unit_body(u, carry):
        a0 = pl.multiple_of(atom_off + u * AU, 8)
        pltpu.sync_copy(idx_hbm.at[pl.ds(a0, AU)], vidx)
        pltpu.async_copy(src_hbm.at[vidx], vg, sem).wait()

        def cyc_body(jc, c2):
            r0 = jc * L
            for k in range(width // 16):
                sl = pl.ds(k * 16, 16)
                g = [vg[r0 + i, sl] for i in range(L)]
                if do_sums:
                    s = g[0] + g[1]
                    for i in range(2, L):
                        s = s + g[i]
                    s = s * 2.0
                    so = pl.ds(width + k * 16, 16)
                    for i in range(L):
                        vout[r0 + i, so] = s
                for i in range(L):
                    x = g[i] + g[(i - 1) % L]
                    if do_sums:
                        vout[r0 + i, sl] = x
                    else:
                        vg[r0 + i, sl] = x
            return c2

        lax.fori_loop(0, ncyc, cyc_body, 0)
        out_src = vout if do_sums else vg
        pltpu.sync_copy(out_src, out_hbm.at[pl.ds(a0, AU)])
        return carry

    lax.fori_loop(u_lo, u_hi, unit_body, 0)


@functools.partial(
    pl.kernel, mesh=_mesh,
    out_type=jax.ShapeDtypeStruct((N, 2 * H), jnp.float32),
    scratch_types=[
        pltpu.VMEM((240, H), jnp.float32),
        pltpu.VMEM((240, 2 * H), jnp.float32),
        pltpu.VMEM((240,), jnp.int32),
        pltpu.SemaphoreType.DMA,
    ],
    compiler_params=_sc_params,
)
def _kernel_a(edge_hbm, cur_hbm, dep_hbm, lift_hbm, vg, vout, vidx, sem):
    del dep_hbm
    wid = _wid()
    _cycle_phase(edge_hbm, cur_hbm, lift_hbm, vg, vout, vidx, sem,
                 L=5, ncyc=48, n_units=C5 // 48, atom_off=0,
                 width=H, do_sums=True, wid=wid)
    _cycle_phase(edge_hbm, cur_hbm, lift_hbm, vg, vout, vidx, sem,
                 L=6, ncyc=40, n_units=C6 // 40, atom_off=N5,
                 width=H, do_sums=True, wid=wid)


@functools.partial(
    pl.kernel, mesh=_mesh,
    out_type=(jax.ShapeDtypeStruct((N, H), jnp.float32),
              jax.ShapeDtypeStruct((N, H), jnp.float32)),
    scratch_types=[
        pltpu.VMEM((240, H), jnp.float32),
        pltpu.VMEM((240, H), jnp.float32),
        pltpu.VMEM((240,), jnp.int32),
        pltpu.SemaphoreType.DMA,
        pltpu.SemaphoreType.DMA,
    ],
    compiler_params=_sc_params,
)
def _kernel_c(ilo_hbm, ihi_hbm, cur_hbm, dep_hbm, lmlo_hbm, lmhi_hbm,
              vglo, vghi, vidx, semlo, semhi):
    del dep_hbm
    wid = _wid()
    for args in ((ilo_hbm, lmlo_hbm, vglo, semlo),
                 (ihi_hbm, lmhi_hbm, vghi, semhi)):
        src, out, vg, sem = args
        _cycle_phase(src, cur_hbm, out, vg, None, vidx, sem,
                     L=5, ncyc=48, n_units=C5 // 48, atom_off=0,
                     width=H, do_sums=False, wid=wid)
        _cycle_phase(src, cur_hbm, out, vg, None, vidx, sem,
                     L=6, ncyc=40, n_units=C6 // 40, atom_off=N5,
                     width=H, do_sums=False, wid=wid)


@functools.partial(
    pl.kernel, mesh=_mesh,
    out_type=(
        jax.ShapeDtypeStruct((BINSZ + BIN_TAIL,), jnp.int32),
        jax.ShapeDtypeStruct((BINSZ + BIN_TAIL,), jnp.int32),
        jax.ShapeDtypeStruct((NW, 96), jnp.int32),
        jax.ShapeDtypeStruct((NW, 96), jnp.int32),
    ),
    scratch_types=[
        pltpu.VMEM((PER_W,), jnp.int32),
        pltpu.VMEM((CAP_W,), jnp.int32),
        pltpu.VMEM((CAP_W,), jnp.int32),
        pltpu.VMEM((96,), jnp.int32),
        pltpu.VMEM((96,), jnp.int32),
        pltpu.VMEM((96,), jnp.int32),
        pltpu.VMEM((96,), jnp.int32),
    ],
    compiler_params=_sc_params,
)
def _kernel_b1(dbl_hbm, dep_hbm, bsrc_hbm, btgt_hbm, tstart_hbm, tnv_hbm,
               vtg, bsrc, btgt, hist, off, gstart, nv16):
    del dep_hbm
    wid = _wid()
    j0 = pl.multiple_of(wid * PER_W, 8)
    pltpu.sync_copy(dbl_hbm.at[pl.ds(j0, PER_W)], vtg)
    iota = lax.iota(jnp.int32, 16)
    zero16 = jnp.zeros((16,), jnp.int32)
    for k in range(6):
        hist[pl.ds(k * 16, 16)] = zero16

    eful = jnp.full((16,), E, jnp.int32)

    def fill_body(v, carry):
        bsrc[pl.ds(v * 16, 16)] = iota + v * 16
        btgt[pl.ds(v * 16, 16)] = eful
        return carry

    lax.fori_loop(0, CAP_W // 16, fill_body, 0)

    def hist_body(v, carry):
        t = vtg[pl.ds(v * 16, 16)]
        b = t // CHB
        cnt, last = plsc.scan_count(b)
        plsc.addupdate_scatter(hist, [b], cnt, mask=last)
        return carry

    lax.fori_loop(0, PER_W // 16, hist_body, 0)

    carry = jnp.zeros((), jnp.int32)
    for k in range(6):
        h = hist[pl.ds(k * 16, 16)]
        hp = ((h + 15) // 16) * 16
        inc = plsc.cumsum(hp)
        exc = inc - hp + carry
        off[pl.ds(k * 16, 16)] = exc
        gstart[pl.ds(k * 16, 16)] = exc + wid * CAP_W
        nv16[pl.ds(k * 16, 16)] = (h + 15) // 16
        carry = inc[15] + carry

    pltpu.sync_copy(gstart, tstart_hbm.at[wid])
    pltpu.sync_copy(nv16, tnv_hbm.at[wid])

    def bin_body(v, carry2):
        t = vtg[pl.ds(v * 16, 16)]
        b = t // CHB
        cnt, last = plsc.scan_count(b)
        o = plsc.load_gather(off, [b])
        pos = o + cnt - 1
        jg = j0 + v * 16 + iota
        src = jnp.where(jg >= N, jg - N, jg)
        plsc.store_scatter(bsrc, [pos], src)
        plsc.store_scatter(btgt, [pos], t)
        plsc.store_scatter(off, [b], pos + 1, mask=last)
        return carry2

    lax.fori_loop(0, PER_W // 16, bin_body, 0)

    for k in range(6):
        bid = iota + k * 16
        ends = off[pl.ds(k * 16, 16)]
        h = hist[pl.ds(k * 16, 16)]
        npad = (16 - (h & 15)) & 15
        dsrc = (bid * 7 + wid * 53) % N
        dtgt = jnp.full((16,), E, jnp.int32)
        for p in range(15):
            m = npad > p
            plsc.store_scatter(bsrc, [ends + p], dsrc, mask=m)
            plsc.store_scatter(btgt, [ends + p], dtgt, mask=m)

    wcap = pl.multiple_of(wid * CAP_W, 8)
    pltpu.sync_copy(bsrc, bsrc_hbm.at[pl.ds(wcap, CAP_W)])
    pltpu.sync_copy(btgt, btgt_hbm.at[pl.ds(wcap, CAP_W)])

    @pl.when(wid == NW - 1)
    def _():
        for v in range(BIN_TAIL // 16):
            bsrc[pl.ds(v * 16, 16)] = iota + v * 16
            btgt[pl.ds(v * 16, 16)] = jnp.full((16,), E, jnp.int32)
        pltpu.sync_copy(bsrc.at[pl.ds(0, BIN_TAIL)],
                        bsrc_hbm.at[pl.ds(BINSZ, BIN_TAIL)])
        pltpu.sync_copy(btgt.at[pl.ds(0, BIN_TAIL)],
                        btgt_hbm.at[pl.ds(BINSZ, BIN_TAIL)])


def _make_b2(ntab, chp, acc_rows, trows, bpc):
    nchunk = E // chp
    width = H

    @functools.partial(
        pl.kernel, mesh=_mesh,
        out_type=tuple(jax.ShapeDtypeStruct((E, width), jnp.float32)
                       for _ in range(ntab)),
        scratch_types=(
            [pltpu.VMEM((B2B,), jnp.int32)] * 3
            + [pltpu.VMEM((B2B, width), jnp.float32)] * ntab
            + [pltpu.VMEM((128, width), jnp.float32)]
            + [pltpu.VMEM((NW, 96), jnp.int32)] * 2
            + [pltpu.VMEM_SHARED((acc_rows, width), jnp.float32)] * ntab
            + [pltpu.SemaphoreType.DMA] * (2 * ntab)
        ),
        compiler_params=_sc_params,
    )
    def b2(*args):
        (bsrc_hbm, btgt_hbm, tstart_hbm, tnv_hbm, dep_hbm), rest = (
            args[:5], args[5:])
        del dep_hbm
        srcs, rest = rest[:ntab], rest[ntab:]
        outs, rest = rest[:ntab], rest[ntab:]
        (msrc, mtgt, loc), rest = rest[:3], rest[3:]
        rows, rest = rest[:ntab], rest[ntab:]
        (zbuf, vts, vtn), rest = rest[:3], rest[3:]
        accs, rest = rest[:ntab], rest[ntab:]
        semg, sems = rest[:ntab], rest[ntab:]
        sc = lax.axis_index("c")
        t = lax.axis_index("s")
        iota = lax.iota(jnp.int32, 16)
        zf = jnp.zeros((16,), jnp.float32)
        for r in range(128):
            for kk in range(width // 16):
                zbuf[r, pl.ds(kk * 16, 16)] = zf
        pltpu.sync_copy(tstart_hbm, vts)
        pltpu.sync_copy(tnv_hbm, vtn)

        r0 = pl.multiple_of(t * trows, 8)

        def zero_acc():
            for a in accs:
                for off in range(0, trows, 128):
                    pltpu.sync_copy(zbuf, a.at[pl.ds(r0 + off, 128)])

        zero_acc()
        plsc.subcore_barrier()

        def chunk_body(it, carry):
            c = it * NC + sc
            base = pl.multiple_of(c * chp, 8)
            b0 = c * bpc
            for wsub in range(2):
                w = t * 2 + wsub
                sv = vts[w, pl.ds(b0, 16)]
                nvv = vtn[w, pl.ds(b0, 16)]
                start = sv[0]
                if bpc == 2:
                    nv = (sv[1] - sv[0]) // 16 + nvv[1]
                else:
                    nv = nvv[0]
                nb = (nv + (B2V - 1)) // B2V

                def batch_body(kb, bc):
                    pos = pl.multiple_of(start + kb * B2B, 8)
                    rem = nv * 16 - kb * B2B
                    pltpu.sync_copy(bsrc_hbm.at[pl.ds(pos, B2B)], msrc)
                    pltpu.sync_copy(btgt_hbm.at[pl.ds(pos, B2B)], mtgt)
                    for v in range(B2V):
                        tg = mtgt[pl.ds(v * 16, 16)]
                        lo = tg - base
                        ok = (lo >= 0) & (lo < chp) & ((iota + v * 16) < rem)
                        loc[pl.ds(v * 16, 16)] = jnp.where(ok, lo, chp)
                    gd = [pltpu.async_copy(srcs[n].at[msrc], rows[n],
                                           semg[n]) for n in range(ntab)]
                    for n in range(ntab):
                        gd[n].wait()
                    sd = [pltpu.async_copy(rows[n], accs[n].at[loc],
                                           sems[n], add=True)
                          for n in range(ntab)]
                    for n in range(ntab):
                        sd[n].wait()
                    return bc

                lax.fori_loop(0, nb, batch_body, 0)
            plsc.subcore_barrier()

            for n in range(ntab):
                @pl.when(t < NS - 1)
                def _(n=n):
                    pltpu.sync_copy(accs[n].at[pl.ds(r0, trows)],
                                    outs[n].at[pl.ds(base + r0, trows)])

                @pl.when(t == NS - 1)
                def _(n=n):
                    last = chp - (NS - 1) * trows
                    pltpu.sync_copy(accs[n].at[pl.ds(r0, last)],
                                    outs[n].at[pl.ds(base + r0, last)])

            zero_acc()
            plsc.subcore_barrier()
            return carry

        lax.fori_loop(0, nchunk // NC, chunk_body, 0)

    return b2


_B2Y = _make_b2(1, 4000, 4096, 256, 2)
_B2I = _make_b2(2, 2000, 2048, 128, 1)


def _m1_body(l_ref, clo_ref, chi_ref, w_ref, o_ref):
    w = w_ref[...]
    acc = jnp.dot(l_ref[...], w[0:2 * H, :], preferred_element_type=jnp.float32)
    acc += jnp.dot(clo_ref[...], w[2 * H:3 * H, :],
                   preferred_element_type=jnp.float32)
    acc += jnp.dot(chi_ref[...], w[3 * H:, :],
                   preferred_element_type=jnp.float32)
    o_ref[...] = jnp.maximum(acc, 0.0)


def _m2_body(a_ref, b_ref, w_ref, e1_ref, e2_ref, o_ref):
    x = a_ref[...] * (1.0 + e1_ref[0, 0]) + b_ref[...] * (1.0 + e2_ref[0, 0])
    acc = jnp.dot(x, w_ref[...], preferred_element_type=jnp.float32)
    o_ref[...] = jnp.maximum(acc, 0.0)


def _m3_body(lmlo_ref, lmhi_ref, lf_ref, w_ref, e_ref, o_ref):
    w = w_ref[...]
    lf = lf_ref[...]
    g = 1.0 + e_ref[0, 0]
    xlo = lmlo_ref[...] * g + lf[:, 0:H]
    xhi = lmhi_ref[...] * g + lf[:, H:]
    acc = jnp.dot(xlo, w[0:H, :], preferred_element_type=jnp.float32)
    acc += jnp.dot(xhi, w[H:, :], preferred_element_type=jnp.float32)
    o_ref[...] = jnp.maximum(acc, 0.0)


def _row_spec(rows, cols):
    return pl.BlockSpec((rows, cols), lambda i: (i, 0))


def _full_spec(r, c):
    return pl.BlockSpec((r, c), lambda i: (0, 0))


_M1 = pl.pallas_call(
    _m1_body,
    grid=(N // 528,),
    in_specs=[_row_spec(528, 2 * H), _row_spec(528, H), _row_spec(528, H),
              _full_spec(4 * H, H)],
    out_specs=_row_spec(528, H),
    out_shape=jax.ShapeDtypeStruct((N, H), jnp.float32),
)

_M2 = pl.pallas_call(
    _m2_body,
    grid=(E // 640,),
    in_specs=[_row_spec(640, H), _row_spec(640, H), _full_spec(H, H),
              _full_spec(1, 1), _full_spec(1, 1)],
    out_specs=_row_spec(640, H),
    out_shape=jax.ShapeDtypeStruct((E, H), jnp.float32),
)

_M3 = pl.pallas_call(
    _m3_body,
    grid=(N // 528,),
    in_specs=[_row_spec(528, H), _row_spec(528, H), _row_spec(528, 2 * H),
              _full_spec(2 * H, 2 * H), _full_spec(1, 1)],
    out_specs=_row_spec(528, 2 * H),
    out_shape=jax.ShapeDtypeStruct((N, 2 * H), jnp.float32),
)


def kernel(edge_rep, cycle_rep, e5_idx, e6_idx, W_lift, W_lvl1, W_lvl2,
           eps1_1, eps1_2, eps2):
    e5 = e5_idx.astype(jnp.int32)
    e6 = e6_idx.astype(jnp.int32)
    cur = jnp.concatenate([e5.reshape(-1), e6.reshape(-1)])
    prev = jnp.concatenate([jnp.roll(e5, 1, axis=1).reshape(-1),
                            jnp.roll(e6, 1, axis=1).reshape(-1)])
    dbl = jnp.concatenate([cur, prev,
                           jnp.full((JP - J,), E, jnp.int32)])

    cr_lo = cycle_rep[:, :H]
    cr_hi = cycle_rep[:, H:]
    bsrc, btgt, tstart, tnv = _kernel_b1(dbl, cur[0:8])
    lift = _kernel_a(edge_rep, cur, tstart[0:8, 0:8])
    y = _M1(lift, cr_lo, cr_hi, W_lvl1)
    ilo, ihi = _B2I(bsrc, btgt, tstart, tnv, lift[0:8, 0:8], cr_lo, cr_hi)
    (lvl,) = _B2Y(bsrc, btgt, tstart, tnv, ilo[0:8, 0:8], y)
    lm_lo, lm_hi = _kernel_c(ilo, ihi, cur, lvl[0:8, 0:8])
    e11 = eps1_1.reshape(1, 1).astype(jnp.float32)
    e12 = eps1_2.reshape(1, 1).astype(jnp.float32)
    e2 = eps2.reshape(1, 1).astype(jnp.float32)
    edge_out = _M2(edge_rep, lvl, W_lvl2, e11, e12)
    cycle_out = _M3(lm_lo, lm_hi, lift, W_lift, e2)
    return (edge_out, cycle_out)
